# baseline (speedup 1.0000x reference)
import functools

import jax
import jax.numpy as jnp
from jax import lax
from jax.experimental import pallas as pl
from jax.experimental.pallas import tpu as pltpu
from jax.experimental.pallas import tpu_sc as plsc

N = 10000
E = 320000
H = 4
C = 128
HC = H * C
NG = 16

NWORK = 32
PER_W = E // NWORK
GW = 80
NWIN = PER_W // GW

NPADN = 10016
NT = NPADN // 32
NB = 10256
SW = 128
NSW = E // SW
AGW = 128
EPAD = E + AGW


def _proj_kernel(x_ref, w_ref, b_ref, o_ref):
    o_ref[...] = jnp.dot(x_ref[...], w_ref[...],
                         preferred_element_type=jnp.float32) + b_ref[...]


def _proj(x, w, b):
    m, _ = x.shape
    n = w.shape[1]
    return pl.pallas_call(
        _proj_kernel,
        out_shape=jax.ShapeDtypeStruct((m, n), jnp.float32),
    )(x, w, b[None, :])


def _gather_body(table_hbm, idx_hbm, out_hbm, idx_v, rows_v, gsem):
    wid = lax.axis_index("s") * 2 + lax.axis_index("c")
    base = wid * PER_W

    @pl.loop(0, NWIN)
    def _(w):
        off = base + w * GW
        pltpu.sync_copy(idx_hbm.at[pl.ds(off, GW)], idx_v)
        pltpu.async_copy(table_hbm.at[idx_v], rows_v, gsem).wait()
        pltpu.sync_copy(rows_v, out_hbm.at[pl.ds(off, GW)])


def _gather_rows(table, idx, d):
    mesh = plsc.VectorSubcoreMesh(core_axis_name="c", subcore_axis_name="s")
    f = pl.kernel(
        _gather_body,
        out_type=jax.ShapeDtypeStruct((E, d), jnp.float32),
        mesh=mesh,
        scratch_types=[
            pltpu.VMEM((GW,), jnp.int32),
            pltpu.VMEM((GW, d), jnp.float32),
            pltpu.SemaphoreType.DMA,
        ],
        name=f"sc_gather_{d}",
    )
    return f(table, idx)


def _sort_body(dst_hbm, eye_hbm, iota_hbm, perm_hbm, offs_hbm,
               histv, tmpv, wbuf, posv, idsv, eyev, iotav,
               hist_sh, base_sh, sem):
    c = lax.axis_index("c")
    s = lax.axis_index("s")

    @pl.when(c == 0)
    def _():
        pltpu.sync_copy(eye_hbm, eyev)
        pltpu.sync_copy(iota_hbm, iotav)
        eyerows = [eyev[l, :] for l in range(16)]
        inc0 = eyerows[0]
        zerov = inc0 * 0
        iov = iotav[...]
        nwin = (NSW - s + 15) // 16

        @pl.loop(0, NB // 16)
        def _(i):
            histv[pl.ds(i * 16, 16)] = zerov

        def hstep(k, carry):
            off = (s + k * 16) * SW
            pltpu.sync_copy(dst_hbm.at[pl.ds(off, SW)], wbuf)
            for a in range(SW // 16):
                dvec = wbuf[pl.ds(a * 16, 16)]
                for l in range(16):
                    d = dvec[l]
                    histv[pl.ds(d, 16)] = histv[pl.ds(d, 16)] + inc0
            return carry

        lax.fori_loop(0, nwin, hstep, 0)
        pltpu.sync_copy(histv, hist_sh.at[pl.ds(s * NB, NB)])
        plsc.subcore_barrier()

        @pl.when(s == 0)
        def _():
            @pl.loop(0, NB // 16)
            def _(i):
                histv[pl.ds(i * 16, 16)] = zerov

            for tt in range(16):
                pltpu.sync_copy(hist_sh.at[pl.ds(tt * NB, NB)], tmpv)

                @pl.loop(0, NB // 16)
                def _(i):
                    histv[pl.ds(i * 16, 16)] = (histv[pl.ds(i * 16, 16)]
                                                + tmpv[pl.ds(i * 16, 16)])

            def scanstep(i, run):
                v = histv[pl.ds(i * 16, 16)]
                acc = run
                exv = zerov
                for l in range(16):
                    exv = exv + eyerows[l] * acc
                    acc = acc + v[l]
                tmpv[pl.ds(i * 16, 16)] = exv
                return acc

            lax.fori_loop(0, NB // 16, scanstep, 0)
            pltpu.sync_copy(tmpv, offs_hbm)
            for tt in range(16):
                pltpu.sync_copy(tmpv, base_sh.at[pl.ds(tt * NB, NB)])
                pltpu.sync_copy(hist_sh.at[pl.ds(tt * NB, NB)], histv)

                @pl.loop(0, NB // 16)
                def _(i):
                    tmpv[pl.ds(i * 16, 16)] = (tmpv[pl.ds(i * 16, 16)]
                                               + histv[pl.ds(i * 16, 16)])
        plsc.subcore_barrier()

        pltpu.sync_copy(base_sh.at[pl.ds(s * NB, NB)], histv)

        def pstep(k, carry):
            off = (s + k * 16) * SW
            pltpu.sync_copy(dst_hbm.at[pl.ds(off, SW)], wbuf)
            for a in range(SW // 16):
                dvec = wbuf[pl.ds(a * 16, 16)]
                pvec = zerov
                for l in range(16):
                    d = dvec[l]
                    bv = histv[pl.ds(d, 16)]
                    histv[pl.ds(d, 16)] = bv + inc0
                    pvec = pvec + eyerows[l] * bv[0]
                posv[0, pl.ds(a * 16, 16)] = pvec
                idsv[0, pl.ds(a * 16, 16)] = iov + (off + a * 16)
            pltpu.sync_copy(idsv.at[0], perm_hbm.at[posv.at[0]])
            return carry

        lax.fori_loop(0, nwin, pstep, 0)


def _sort_edges(dst, eye, iota):
    mesh = plsc.VectorSubcoreMesh(core_axis_name="c", subcore_axis_name="s")
    f = pl.kernel(
        _sort_body,
        out_type=(jax.ShapeDtypeStruct((E,), jnp.int32),
                  jax.ShapeDtypeStruct((NB,), jnp.int32)),
        mesh=mesh,
        scratch_types=[
            pltpu.VMEM((NB,), jnp.int32),
            pltpu.VMEM((NB,), jnp.int32),
            pltpu.VMEM((SW,), jnp.int32),
            pltpu.VMEM((1, SW), jnp.int32),
            pltpu.VMEM((1, SW), jnp.int32),
            pltpu.VMEM((16, 16), jnp.int32),
            pltpu.VMEM((16,), jnp.int32),
            pltpu.VMEM_SHARED((16 * NB,), jnp.int32),
            pltpu.VMEM_SHARED((16 * NB,), jnp.int32),
            pltpu.SemaphoreType.DMA,
        ],
        name="sc_sort_by_dst",
    )
    return f(dst, eye, iota)


def _permute_body(perm_hbm, src_hbm, dst_hbm, eat_hbm,
                  srcs_hbm, dsts_hbm, eas_hbm,
                  permv, idxcv, srcv, dstv, eav, sem):
    wid = lax.axis_index("s") * 2 + lax.axis_index("c")
    nwin = (NSW - wid + 31) // 32

    def step(k, carry):
        off = pl.multiple_of((wid + k * 32) * SW, SW)
        pltpu.sync_copy(perm_hbm.at[pl.ds(off, SW)], permv.at[0])
        pltpu.async_copy(src_hbm.at[permv.at[0]], srcv, sem).wait()
        pltpu.async_copy(dst_hbm.at[permv.at[0]], dstv, sem).wait()
        pltpu.sync_copy(srcv, srcs_hbm.at[pl.ds(off, SW)])
        pltpu.sync_copy(dstv, dsts_hbm.at[pl.ds(off, SW)])
        for cc in range(4):
            for a in range(SW // 16):
                idxcv[0, pl.ds(a * 16, 16)] = (permv[0, pl.ds(a * 16, 16)]
                                               + cc * E)
            pltpu.async_copy(eat_hbm.at[idxcv.at[0]], eav, sem).wait()
            pltpu.sync_copy(eav, eas_hbm.at[pl.ds(pl.multiple_of(cc * E + off, SW), SW)])
        return carry

    lax.fori_loop(0, nwin, step, 0)


def _permute_payload(perm, src, dst, eatf):
    mesh = plsc.VectorSubcoreMesh(core_axis_name="c", subcore_axis_name="s")
    f = pl.kernel(
        _permute_body,
        out_type=(jax.ShapeDtypeStruct((E,), jnp.int32),
                  jax.ShapeDtypeStruct((E,), jnp.int32),
                  jax.ShapeDtypeStruct((4 * E,), jnp.float32)),
        mesh=mesh,
        scratch_types=[
            pltpu.VMEM((1, SW), jnp.int32),
            pltpu.VMEM((1, SW), jnp.int32),
            pltpu.VMEM((SW,), jnp.int32),
            pltpu.VMEM((SW,), jnp.int32),
            pltpu.VMEM((SW,), jnp.float32),
            pltpu.SemaphoreType.DMA,
        ],
        name="sc_permute_payload",
    )
    return f(perm, src, dst, eatf)


def _alpha_kernel(gl_ref, gr_ref, ea_ref, we_ref, a_ref, o_ref):
    e = lax.dot_general(ea_ref[...], we_ref[...],
                        dimension_numbers=(((0,), (0,)), ((), ())),
                        preferred_element_type=jnp.float32)
    m = gl_ref[...] + gr_ref[...] + e
    m = jnp.where(m >= 0.0, m, 0.2 * m)
    o_ref[...] = jnp.dot(m, a_ref[...], preferred_element_type=jnp.float32)


def _alpha(gl, gr, easT, WeP, A):
    BE = 2048
    grid = (E + BE - 1) // BE
    return pl.pallas_call(
        _alpha_kernel,
        out_shape=jax.ShapeDtypeStruct((E, H), jnp.float32),
        grid=(grid,),
        in_specs=[
            pl.BlockSpec((BE, HC), lambda i: (i, 0)),
            pl.BlockSpec((BE, HC), lambda i: (i, 0)),
            pl.BlockSpec((4, BE), lambda i: (0, i)),
            pl.BlockSpec((4, HC), lambda i: (0, 0)),
            pl.BlockSpec((HC, H), lambda i: (0, 0)),
        ],
        out_specs=pl.BlockSpec((BE, H), lambda i: (i, 0)),
    )(gl, gr, easT, WeP, A)


AGW2 = 96
DWP = AGW2 + 16


def _agg_body(xl_hbm, srcs_hbm, dsts_hbm, p16_hbm, meta_hbm, fz_hbm,
              wsum_hbm, den_hbm,
              metav, fzv, srcw0, srcw1, rows0, rows1, dstwf, pvff,
              accv, denv, gsem0, gsem1, osem):
    wid = lax.axis_index("s") * 2 + lax.axis_index("c")
    pltpu.sync_copy(fz_hbm, fzv)
    zerof = fzv[...]
    pltpu.sync_copy(meta_hbm.at[pl.ds(pl.multiple_of(wid * 8, 8), 8)],
                    metav.at[pl.ds(0, 8)])
    mv = metav[pl.ds(0, 16)]
    estart = mv[0]
    eend = mv[1]
    nlo = wid * NT
    astart = pl.multiple_of((estart // 8) * 8, 8)
    nwin = (eend - astart + AGW2 - 1) // AGW2

    @pl.loop(0, NT)
    def _(i):
        denv[pl.ds(i * 16, 16)] = zerof

    bufs = ((srcw0, rows0, gsem0, 0), (srcw1, rows1, gsem1, 1))

    def start(w, b):
        srcw, rows, gsem, bi = bufs[b]
        base = pl.multiple_of(astart + w * AGW2, 8)
        pltpu.sync_copy(srcs_hbm.at[pl.ds(base, AGW2)], srcw.at[0])
        pltpu.sync_copy(dsts_hbm.at[pl.ds(base, AGW2)],
                        dstwf.at[pl.ds(bi * DWP, AGW2)])
        pltpu.sync_copy(
            p16_hbm.at[pl.ds(pl.multiple_of(base * 16, 128), AGW2 * 16)],
            pvff.at[pl.ds(bi * AGW2 * 16, AGW2 * 16)])
        pltpu.async_copy(xl_hbm.at[srcw.at[0]], rows, gsem)

    @pl.when(nwin > 0)
    def _():
        start(0, 0)

    @pl.when(nwin > 1)
    def _():
        start(1, 1)

    def process(w, b, carry):
        srcw, rows, gsem, bi = bufs[b]
        valid = w < nwin

        @pl.when(valid)
        def _():
            pltpu.make_async_copy(xl_hbm.at[srcw.at[0]], rows, gsem).wait()
        base = pl.multiple_of(astart + w * AGW2, 8)
        jlo = jnp.maximum(0, estart - base)
        jhi = jnp.minimum(AGW2, eend - base)
        jhi = jnp.where(valid, jhi, jlo)
        jhi = jnp.maximum(jlo, jhi)

        def edge_step(j, ecarry):
            cur, flip, cnt = ecarry[0], ecarry[1], ecarry[2]
            accs = ecarry[3:]
            d = dstwf[pl.ds(bi * DWP + j, 16)][0]
            pev = pvff[pl.ds(bi * AGW2 * 16 + j * 16, 16)]
            is_new = d != cur

            @pl.when(is_new)
            def _():
                @pl.when(cnt > 0)
                def _():
                    pltpu.make_async_copy(
                        accv.at[pl.ds(0, HC)],
                        wsum_hbm.at[pl.ds(0, HC)], osem).wait()
                fo = pl.multiple_of(flip, 8)
                for k in range(HC // 16):
                    accv[pl.ds(fo + k * 16, 16)] = accs[k]
                pltpu.async_copy(
                    accv.at[pl.ds(fo, HC)],
                    wsum_hbm.at[pl.ds(pl.multiple_of(cur * HC, 8), HC)],
                    osem)

            keepf = jnp.where(is_new, 0.0, 1.0)
            cur = jnp.where(is_new, d, cur)
            flip = jnp.where(is_new, HC - flip, flip)
            cnt = cnt + jnp.where(is_new, 1, 0)
            doff = (d - nlo) * 16
            denv[pl.ds(doff, 16)] = denv[pl.ds(doff, 16)] + pev
            ws = (pev[0], pev[1], pev[2], pev[3])
            naccs = tuple(
                accs[k] * keepf + rows[j, pl.ds(k * 16, 16)] * ws[k // 8]
                for k in range(HC // 16))
            return (cur, flip, cnt) + naccs

        carry = lax.fori_loop(jlo, jhi, edge_step, carry)

        @pl.when(w + 2 < nwin)
        def _():
            start(w + 2, b)
        return carry

    carry0 = (nlo, 0, 0) + tuple(zerof for _ in range(HC // 16))

    def pair(w2, carry):
        for b in (0, 1):
            carry = process(w2 * 2 + b, b, carry)
        return carry

    carry = lax.fori_loop(0, (nwin + 1) // 2, pair, carry0)
    cur, flip, cnt = carry[0], carry[1], carry[2]
    accs = carry[3:]

    @pl.when(cnt > 0)
    def _():
        pltpu.make_async_copy(accv.at[pl.ds(0, HC)],
                              wsum_hbm.at[pl.ds(0, HC)], osem).wait()
    fo = pl.multiple_of(flip, 8)
    for k in range(HC // 16):
        accv[pl.ds(fo + k * 16, 16)] = accs[k]
    pltpu.sync_copy(accv.at[pl.ds(fo, HC)],
                    wsum_hbm.at[pl.ds(pl.multiple_of(cur * HC, 8), HC)])
    pltpu.sync_copy(denv, den_hbm.at[pl.ds(
        pl.multiple_of(wid * NT * 16, 16), NT * 16)])


def _aggregate(xl, srcs_p, dsts_p, p16f, meta, fz):
    mesh = plsc.VectorSubcoreMesh(core_axis_name="c", subcore_axis_name="s")
    f = pl.kernel(
        _agg_body,
        out_type=(jax.ShapeDtypeStruct((NPADN * HC,), jnp.float32),
                  jax.ShapeDtypeStruct((NPADN * 16,), jnp.float32)),
        mesh=mesh,
        scratch_types=[
            pltpu.VMEM((16,), jnp.int32),
            pltpu.VMEM((16,), jnp.float32),
            pltpu.VMEM((1, AGW2), jnp.int32),
            pltpu.VMEM((1, AGW2), jnp.int32),
            pltpu.VMEM((AGW2, HC), jnp.float32),
            pltpu.VMEM((AGW2, HC), jnp.float32),
            pltpu.VMEM((2 * DWP,), jnp.int32),
            pltpu.VMEM((2 * AGW2 * 16,), jnp.float32),
            pltpu.VMEM((2 * HC,), jnp.float32),
            pltpu.VMEM((NT * 16,), jnp.float32),
            pltpu.SemaphoreType.DMA,
            pltpu.SemaphoreType.DMA,
            pltpu.SemaphoreType.DMA,
        ],
        name="sc_sorted_agg",
    )
    return f(xl, srcs_p, dsts_p, p16f, meta, fz)


def _expand_body(xrf_hbm, dsts_hbm, meta_hbm, gr_hbm,
                 metav, dstwf, out0, out1, rowb, wsem0, wsem1):
    wid = lax.axis_index("s") * 2 + lax.axis_index("c")
    pltpu.sync_copy(meta_hbm.at[pl.ds(pl.multiple_of(wid * 8, 8), 8)],
                    metav.at[pl.ds(0, 8)])
    mv = metav[pl.ds(0, 16)]
    estart = mv[0]
    eend = mv[1]
    astart = pl.multiple_of((estart // 8) * 8, 8)
    nwin = (eend - astart + AGW2 - 1) // AGW2
    bufs = ((out0, wsem0), (out1, wsem1))

    def process(w, b, cur):
        out, wsem = bufs[b]
        valid = w < nwin
        base = pl.multiple_of(astart + w * AGW2, 8)
        pltpu.sync_copy(dsts_hbm.at[pl.ds(base, AGW2)],
                        dstwf.at[pl.ds(0, AGW2)])

        @pl.when(valid & (w >= 2))
        def _():
            pltpu.make_async_copy(
                out, gr_hbm.at[pl.ds(0, AGW2 * HC)], wsem).wait()

        def edge_step(j, cur):
            d = dstwf[pl.ds(j, 16)][0]

            @pl.when(d != cur)
            def _():
                pltpu.sync_copy(
                    xrf_hbm.at[pl.ds(pl.multiple_of(d * HC, 8), HC)], rowb)
            for k in range(HC // 16):
                out[pl.ds(j * HC + k * 16, 16)] = rowb[pl.ds(k * 16, 16)]
            return d

        jhi = jnp.where(valid, AGW2, 0)
        cur = lax.fori_loop(0, jhi, edge_step, cur)

        @pl.when(valid)
        def _():
            pltpu.async_copy(
                out, gr_hbm.at[pl.ds(pl.multiple_of(base * HC, 8),
                                     AGW2 * HC)], wsem)
        return cur

    def pair(w2, cur):
        for b in (0, 1):
            cur = process(w2 * 2 + b, b, cur)
        return cur

    lax.fori_loop(0, (nwin + 1) // 2, pair, -1)

    @pl.when(nwin > 1)
    def _():
        pltpu.make_async_copy(out0, gr_hbm.at[pl.ds(0, AGW2 * HC)],
                              wsem0).wait()
        pltpu.make_async_copy(out1, gr_hbm.at[pl.ds(0, AGW2 * HC)],
                              wsem1).wait()

    @pl.when(nwin == 1)
    def _():
        pltpu.make_async_copy(out0, gr_hbm.at[pl.ds(0, AGW2 * HC)],
                              wsem0).wait()


def _expand_rows(xrf, dsts_p, meta):
    mesh = plsc.VectorSubcoreMesh(core_axis_name="c", subcore_axis_name="s")
    f = pl.kernel(
        _expand_body,
        out_type=jax.ShapeDtypeStruct((EPAD * HC,), jnp.float32),
        mesh=mesh,
        scratch_types=[
            pltpu.VMEM((16,), jnp.int32),
            pltpu.VMEM((AGW2 + 16,), jnp.int32),
            pltpu.VMEM((AGW2 * HC,), jnp.float32),
            pltpu.VMEM((AGW2 * HC,), jnp.float32),
            pltpu.VMEM((HC,), jnp.float32),
            pltpu.SemaphoreType.DMA,
            pltpu.SemaphoreType.DMA,
        ],
        name="sc_expand_rows",
    )
    return f(xrf, dsts_p, meta)


def _gat_layer(h, srcs, dsts, eas, srcs_p, dsts_p, meta, fz,
               Wl, bl, Wr, br, We, att, bc):
    xl = _proj(h, Wl, bl)
    xr = _proj(h, Wr, br)
    gl = _gather_rows(xl, srcs, HC)
    gr = _expand_rows(xr.reshape(-1), dsts_p, meta).reshape(EPAD, HC)
    A = (att[:, :, None] * jnp.eye(H, dtype=jnp.float32)[:, None, :])
    A = A.reshape(HC, H)
    WeP = jnp.pad(We, ((0, 1), (0, 0)))
    alpha = _alpha(gl, gr, eas.reshape(4, E), WeP, A)
    gmax = jnp.max(alpha, axis=0)
    p = jnp.exp(alpha - gmax[None, :])
    p16 = jnp.pad(p, ((0, AGW), (0, 12))).reshape(-1)
    wsumf, denf = _aggregate(xl, srcs_p, dsts_p, p16, meta, fz)
    wsum = wsumf.reshape(NPADN, HC)[:N].reshape(N, H, C)
    den = denf.reshape(NPADN, 16)[:N, :H]
    out = jnp.where(den[:, :, None] > 0.0,
                    wsum / (den[:, :, None] + 1e-16), 0.0)
    return out.reshape(N, HC) + bc


def kernel(x, edge_index, edge_attr, batch, Wl0, bl0, Wr0, br0, We0, att0,
           bc0, Wl1, bl1, Wr1, br1, We1, att1, bc1, ln_g, ln_b, W1, b1, W2,
           b2, W3, b3):
    src, dst = edge_index[0], edge_index[1]
    eye = jnp.eye(16, dtype=jnp.int32)
    iota = jnp.arange(16, dtype=jnp.int32)
    fz = jnp.zeros((16,), jnp.float32)
    perm, offs = _sort_edges(dst, eye, iota)
    eatf = jnp.pad(edge_attr.T, ((0, 1), (0, 0))).reshape(-1)
    srcs, dsts, eas = _permute_payload(perm, src, dst, eatf)
    srcs_p = jnp.pad(srcs, (0, AGW))
    dsts_p = jnp.pad(dsts, (0, AGW))
    bnd = offs[jnp.arange(33, dtype=jnp.int32) * NT]
    meta = jnp.zeros((32, 8), jnp.int32)
    meta = meta.at[:, 0].set(bnd[:-1]).at[:, 1].set(bnd[1:]).reshape(-1)

    h = _gat_layer(x, srcs, dsts, eas, srcs_p, dsts_p, meta, fz,
                   Wl0, bl0, Wr0, br0, We0, att0, bc0)
    h = jax.nn.relu(h)
    mu = jnp.mean(h, axis=-1, keepdims=True)
    var = jnp.var(h, axis=-1, keepdims=True)
    h = (h - mu) / jnp.sqrt(var + 1e-5) * ln_g + ln_b
    h2 = _gat_layer(h, srcs, dsts, eas, srcs_p, dsts_p, meta, fz,
                    Wl1, bl1, Wr1, br1, We1, att1, bc1)
    emb = h2
    h2 = jax.nn.relu(h2)
    pooled = jax.ops.segment_max(h2, batch, num_segments=NG)
    z = jax.nn.relu(pooled @ W1 + b1)
    z = jax.nn.relu(z @ W2 + b2)
    logits = z @ W3 + b3
    logp = jax.nn.log_softmax(logits, axis=1)
    return (emb, logp)


# speedup vs baseline: 1.1990x; 1.1990x over previous
import functools

import jax
import jax.numpy as jnp
from jax import lax
from jax.experimental import pallas as pl
from jax.experimental.pallas import tpu as pltpu
from jax.experimental.pallas import tpu_sc as plsc

N = 10000
E = 320000
H = 4
C = 128
HC = H * C
NG = 16

NWORK = 32
PER_W = E // NWORK
GW = 80
NWIN = PER_W // GW

NPADN = 10016
NT = NPADN // 32
NB = 10256
SW = 128
NSW = E // SW
AGW = 128
EPAD = E + AGW


def _proj_kernel(x_ref, w_ref, b_ref, o_ref):
    o_ref[...] = jnp.dot(x_ref[...], w_ref[...],
                         preferred_element_type=jnp.float32) + b_ref[...]


def _proj(x, w, b):
    m, _ = x.shape
    n = w.shape[1]
    return pl.pallas_call(
        _proj_kernel,
        out_shape=jax.ShapeDtypeStruct((m, n), jnp.float32),
    )(x, w, b[None, :])


def _gather_body(table_hbm, idx_hbm, out_hbm, idx_v, rows_v, gsem):
    wid = lax.axis_index("s") * 2 + lax.axis_index("c")
    base = wid * PER_W

    @pl.loop(0, NWIN)
    def _(w):
        off = base + w * GW
        pltpu.sync_copy(idx_hbm.at[pl.ds(off, GW)], idx_v)
        pltpu.async_copy(table_hbm.at[idx_v], rows_v, gsem).wait()
        pltpu.sync_copy(rows_v, out_hbm.at[pl.ds(off, GW)])


def _gather_rows(table, idx, d):
    mesh = plsc.VectorSubcoreMesh(core_axis_name="c", subcore_axis_name="s")
    f = pl.kernel(
        _gather_body,
        out_type=jax.ShapeDtypeStruct((E, d), jnp.float32),
        mesh=mesh,
        scratch_types=[
            pltpu.VMEM((GW,), jnp.int32),
            pltpu.VMEM((GW, d), jnp.float32),
            pltpu.SemaphoreType.DMA,
        ],
        name=f"sc_gather_{d}",
    )
    return f(table, idx)


def _sort_body(dst_hbm, eye_hbm, iota_hbm, perm_hbm, offs_hbm,
               histv, tmpv, wbuf, posv, idsv, eyev, iotav,
               hist_sh, base_sh, sem):
    c = lax.axis_index("c")
    s = lax.axis_index("s")

    @pl.when(c == 0)
    def _():
        pltpu.sync_copy(eye_hbm, eyev)
        pltpu.sync_copy(iota_hbm, iotav)
        eyerows = [eyev[l, :] for l in range(16)]
        inc0 = eyerows[0]
        zerov = inc0 * 0
        iov = iotav[...]
        nwin = (NSW - s + 15) // 16

        @pl.loop(0, NB // 16)
        def _(i):
            histv[pl.ds(i * 16, 16)] = zerov

        def hstep(k, carry):
            off = (s + k * 16) * SW
            pltpu.sync_copy(dst_hbm.at[pl.ds(off, SW)], wbuf)
            for a in range(SW // 16):
                dvec = wbuf[pl.ds(a * 16, 16)]
                for l in range(16):
                    d = dvec[l]
                    histv[pl.ds(d, 16)] = histv[pl.ds(d, 16)] + inc0
            return carry

        lax.fori_loop(0, nwin, hstep, 0)
        pltpu.sync_copy(histv, hist_sh.at[pl.ds(s * NB, NB)])
        plsc.subcore_barrier()

        @pl.when(s == 0)
        def _():
            @pl.loop(0, NB // 16)
            def _(i):
                histv[pl.ds(i * 16, 16)] = zerov

            for tt in range(16):
                pltpu.sync_copy(hist_sh.at[pl.ds(tt * NB, NB)], tmpv)

                @pl.loop(0, NB // 16)
                def _(i):
                    histv[pl.ds(i * 16, 16)] = (histv[pl.ds(i * 16, 16)]
                                                + tmpv[pl.ds(i * 16, 16)])

            def scanstep(i, run):
                v = histv[pl.ds(i * 16, 16)]
                acc = run
                exv = zerov
                for l in range(16):
                    exv = exv + eyerows[l] * acc
                    acc = acc + v[l]
                tmpv[pl.ds(i * 16, 16)] = exv
                return acc

            lax.fori_loop(0, NB // 16, scanstep, 0)
            pltpu.sync_copy(tmpv, offs_hbm)
            for tt in range(16):
                pltpu.sync_copy(tmpv, base_sh.at[pl.ds(tt * NB, NB)])
                pltpu.sync_copy(hist_sh.at[pl.ds(tt * NB, NB)], histv)

                @pl.loop(0, NB // 16)
                def _(i):
                    tmpv[pl.ds(i * 16, 16)] = (tmpv[pl.ds(i * 16, 16)]
                                               + histv[pl.ds(i * 16, 16)])
        plsc.subcore_barrier()

        pltpu.sync_copy(base_sh.at[pl.ds(s * NB, NB)], histv)

        def pstep(k, carry):
            off = (s + k * 16) * SW
            pltpu.sync_copy(dst_hbm.at[pl.ds(off, SW)], wbuf)
            for a in range(SW // 16):
                dvec = wbuf[pl.ds(a * 16, 16)]
                pvec = zerov
                for l in range(16):
                    d = dvec[l]
                    bv = histv[pl.ds(d, 16)]
                    histv[pl.ds(d, 16)] = bv + inc0
                    pvec = pvec + eyerows[l] * bv[0]
                posv[0, pl.ds(a * 16, 16)] = pvec
                idsv[0, pl.ds(a * 16, 16)] = iov + (off + a * 16)
            pltpu.sync_copy(idsv.at[0], perm_hbm.at[posv.at[0]])
            return carry

        lax.fori_loop(0, nwin, pstep, 0)


def _sort_edges(dst, eye, iota):
    mesh = plsc.VectorSubcoreMesh(core_axis_name="c", subcore_axis_name="s")
    f = pl.kernel(
        _sort_body,
        out_type=(jax.ShapeDtypeStruct((E,), jnp.int32),
                  jax.ShapeDtypeStruct((NB,), jnp.int32)),
        mesh=mesh,
        scratch_types=[
            pltpu.VMEM((NB,), jnp.int32),
            pltpu.VMEM((NB,), jnp.int32),
            pltpu.VMEM((SW,), jnp.int32),
            pltpu.VMEM((1, SW), jnp.int32),
            pltpu.VMEM((1, SW), jnp.int32),
            pltpu.VMEM((16, 16), jnp.int32),
            pltpu.VMEM((16,), jnp.int32),
            pltpu.VMEM_SHARED((16 * NB,), jnp.int32),
            pltpu.VMEM_SHARED((16 * NB,), jnp.int32),
            pltpu.SemaphoreType.DMA,
        ],
        name="sc_sort_by_dst",
    )
    return f(dst, eye, iota)


def _permute_body(perm_hbm, src_hbm, dst_hbm, eat_hbm,
                  srcs_hbm, dsts_hbm, eas_hbm,
                  permv, idxcv, srcv, dstv, eav, sem):
    wid = lax.axis_index("s") * 2 + lax.axis_index("c")
    nwin = (NSW - wid + 31) // 32

    def step(k, carry):
        off = pl.multiple_of((wid + k * 32) * SW, SW)
        pltpu.sync_copy(perm_hbm.at[pl.ds(off, SW)], permv.at[0])
        pltpu.async_copy(src_hbm.at[permv.at[0]], srcv, sem).wait()
        pltpu.async_copy(dst_hbm.at[permv.at[0]], dstv, sem).wait()
        pltpu.sync_copy(srcv, srcs_hbm.at[pl.ds(off, SW)])
        pltpu.sync_copy(dstv, dsts_hbm.at[pl.ds(off, SW)])
        for cc in range(4):
            for a in range(SW // 16):
                idxcv[0, pl.ds(a * 16, 16)] = (permv[0, pl.ds(a * 16, 16)]
                                               + cc * E)
            pltpu.async_copy(eat_hbm.at[idxcv.at[0]], eav, sem).wait()
            pltpu.sync_copy(eav, eas_hbm.at[pl.ds(pl.multiple_of(cc * E + off, SW), SW)])
        return carry

    lax.fori_loop(0, nwin, step, 0)


def _permute_payload(perm, src, dst, eatf):
    mesh = plsc.VectorSubcoreMesh(core_axis_name="c", subcore_axis_name="s")
    f = pl.kernel(
        _permute_body,
        out_type=(jax.ShapeDtypeStruct((E,), jnp.int32),
                  jax.ShapeDtypeStruct((E,), jnp.int32),
                  jax.ShapeDtypeStruct((4 * E,), jnp.float32)),
        mesh=mesh,
        scratch_types=[
            pltpu.VMEM((1, SW), jnp.int32),
            pltpu.VMEM((1, SW), jnp.int32),
            pltpu.VMEM((SW,), jnp.int32),
            pltpu.VMEM((SW,), jnp.int32),
            pltpu.VMEM((SW,), jnp.float32),
            pltpu.SemaphoreType.DMA,
        ],
        name="sc_permute_payload",
    )
    return f(perm, src, dst, eatf)


def _alpha_kernel(gl_ref, gr_ref, ea_ref, we_ref, a_ref, o_ref):
    e = lax.dot_general(ea_ref[...], we_ref[...],
                        dimension_numbers=(((0,), (0,)), ((), ())),
                        preferred_element_type=jnp.float32)
    m = gl_ref[...] + gr_ref[...] + e
    m = jnp.where(m >= 0.0, m, 0.2 * m)
    o_ref[...] = jnp.dot(m, a_ref[...], preferred_element_type=jnp.float32)


def _alpha(gl, gr, easT, WeP, A):
    BE = 2048
    grid = (E + BE - 1) // BE
    return pl.pallas_call(
        _alpha_kernel,
        out_shape=jax.ShapeDtypeStruct((E, H), jnp.float32),
        grid=(grid,),
        in_specs=[
            pl.BlockSpec((BE, HC), lambda i: (i, 0)),
            pl.BlockSpec((BE, HC), lambda i: (i, 0)),
            pl.BlockSpec((4, BE), lambda i: (0, i)),
            pl.BlockSpec((4, HC), lambda i: (0, 0)),
            pl.BlockSpec((HC, H), lambda i: (0, 0)),
        ],
        out_specs=pl.BlockSpec((BE, H), lambda i: (i, 0)),
    )(gl, gr, easT, WeP, A)


AGW2 = 96
DWP = AGW2 + 16


def _agg_body(xl_hbm, srcs_hbm, dsts_hbm, p16_hbm, meta_hbm, fz_hbm,
              wsum_hbm, den_hbm,
              metav, fzv, srcw0, srcw1, rows0, rows1, dstwf, pvff,
              accv, denv, gsem0, gsem1, osem):
    wid = lax.axis_index("s") * 2 + lax.axis_index("c")
    pltpu.sync_copy(fz_hbm, fzv)
    zerof = fzv[pl.ds(0, 16)]
    gmaxv = fzv[pl.ds(16, 16)]
    pltpu.sync_copy(meta_hbm.at[pl.ds(pl.multiple_of(wid * 8, 8), 8)],
                    metav.at[pl.ds(0, 8)])
    mv = metav[pl.ds(0, 16)]
    estart = mv[0]
    eend = mv[1]
    nlo = wid * NT
    astart = pl.multiple_of((estart // 8) * 8, 8)
    nwin = (eend - astart + AGW2 - 1) // AGW2

    @pl.loop(0, NT)
    def _(i):
        denv[pl.ds(i * 16, 16)] = zerof

    bufs = ((srcw0, rows0, gsem0, 0), (srcw1, rows1, gsem1, 1))

    def start(w, b):
        srcw, rows, gsem, bi = bufs[b]
        base = pl.multiple_of(astart + w * AGW2, 8)
        pltpu.sync_copy(srcs_hbm.at[pl.ds(base, AGW2)], srcw.at[0])
        pltpu.async_copy(dsts_hbm.at[pl.ds(base, AGW2)],
                         dstwf.at[pl.ds(bi * DWP, AGW2)], gsem)
        pltpu.async_copy(
            p16_hbm.at[pl.ds(pl.multiple_of(base * 16, 128), AGW2 * 16)],
            pvff.at[pl.ds(bi * AGW2 * 16, AGW2 * 16)], gsem)
        pltpu.async_copy(xl_hbm.at[srcw.at[0]], rows, gsem)

    @pl.when(nwin > 0)
    def _():
        start(0, 0)

    @pl.when(nwin > 1)
    def _():
        start(1, 1)

    def process(w, b, carry):
        srcw, rows, gsem, bi = bufs[b]
        valid = w < nwin

        base = pl.multiple_of(astart + w * AGW2, 8)

        @pl.when(valid)
        def _():
            pltpu.make_async_copy(
                dsts_hbm.at[pl.ds(base, AGW2)],
                dstwf.at[pl.ds(bi * DWP, AGW2)], gsem).wait()
            pltpu.make_async_copy(
                p16_hbm.at[pl.ds(pl.multiple_of(base * 16, 128), AGW2 * 16)],
                pvff.at[pl.ds(bi * AGW2 * 16, AGW2 * 16)], gsem).wait()
            pltpu.make_async_copy(xl_hbm.at[srcw.at[0]], rows, gsem).wait()
        jlo = jnp.maximum(0, estart - base)
        jhi = jnp.minimum(AGW2, eend - base)
        jhi = jnp.where(valid, jhi, jlo)
        jhi = jnp.maximum(jlo, jhi)

        def edge_step(j, ecarry):
            cur, flip, cnt = ecarry[0], ecarry[1], ecarry[2]
            accs = ecarry[3:]
            d = dstwf[pl.ds(bi * DWP + j, 16)][0]
            av = pvff[pl.ds(bi * AGW2 * 16 + j * 16, 16)]
            pev = jnp.exp(av - gmaxv)
            is_new = d != cur

            @pl.when(is_new)
            def _():
                @pl.when(cnt > 0)
                def _():
                    pltpu.make_async_copy(
                        accv.at[pl.ds(0, HC)],
                        wsum_hbm.at[pl.ds(0, HC)], osem).wait()
                fo = pl.multiple_of(flip, 8)
                for k in range(HC // 16):
                    accv[pl.ds(fo + k * 16, 16)] = accs[k]
                pltpu.async_copy(
                    accv.at[pl.ds(fo, HC)],
                    wsum_hbm.at[pl.ds(pl.multiple_of(cur * HC, 8), HC)],
                    osem)

            keepf = jnp.where(is_new, 0.0, 1.0)
            cur = jnp.where(is_new, d, cur)
            flip = jnp.where(is_new, HC - flip, flip)
            cnt = cnt + jnp.where(is_new, 1, 0)
            doff = (d - nlo) * 16
            denv[pl.ds(doff, 16)] = denv[pl.ds(doff, 16)] + pev
            ws = (pev[0], pev[1], pev[2], pev[3])
            naccs = tuple(
                accs[k] * keepf + rows[j, pl.ds(k * 16, 16)] * ws[k // 8]
                for k in range(HC // 16))
            return (cur, flip, cnt) + naccs

        carry = lax.fori_loop(jlo, jhi, edge_step, carry)

        @pl.when(w + 2 < nwin)
        def _():
            start(w + 2, b)
        return carry

    carry0 = (nlo, 0, 0) + tuple(zerof for _ in range(HC // 16))

    def pair(w2, carry):
        for b in (0, 1):
            carry = process(w2 * 2 + b, b, carry)
        return carry

    carry = lax.fori_loop(0, (nwin + 1) // 2, pair, carry0)
    cur, flip, cnt = carry[0], carry[1], carry[2]
    accs = carry[3:]

    @pl.when(cnt > 0)
    def _():
        pltpu.make_async_copy(accv.at[pl.ds(0, HC)],
                              wsum_hbm.at[pl.ds(0, HC)], osem).wait()
    fo = pl.multiple_of(flip, 8)
    for k in range(HC // 16):
        accv[pl.ds(fo + k * 16, 16)] = accs[k]
    pltpu.sync_copy(accv.at[pl.ds(fo, HC)],
                    wsum_hbm.at[pl.ds(pl.multiple_of(cur * HC, 8), HC)])
    pltpu.sync_copy(denv, den_hbm.at[pl.ds(
        pl.multiple_of(wid * NT * 16, 16), NT * 16)])


def _aggregate(xl, srcs_p, dsts_p, p16f, meta, fz):
    mesh = plsc.VectorSubcoreMesh(core_axis_name="c", subcore_axis_name="s")
    f = pl.kernel(
        _agg_body,
        out_type=(jax.ShapeDtypeStruct((NPADN * HC,), jnp.float32),
                  jax.ShapeDtypeStruct((NPADN * 16,), jnp.float32)),
        mesh=mesh,
        scratch_types=[
            pltpu.VMEM((16,), jnp.int32),
            pltpu.VMEM((32,), jnp.float32),
            pltpu.VMEM((1, AGW2), jnp.int32),
            pltpu.VMEM((1, AGW2), jnp.int32),
            pltpu.VMEM((AGW2, HC), jnp.float32),
            pltpu.VMEM((AGW2, HC), jnp.float32),
            pltpu.VMEM((2 * DWP,), jnp.int32),
            pltpu.VMEM((2 * AGW2 * 16,), jnp.float32),
            pltpu.VMEM((2 * HC,), jnp.float32),
            pltpu.VMEM((NT * 16,), jnp.float32),
            pltpu.SemaphoreType.DMA,
            pltpu.SemaphoreType.DMA,
            pltpu.SemaphoreType.DMA,
        ],
        name="sc_sorted_agg",
    )
    return f(xl, srcs_p, dsts_p, p16f, meta, fz)


def _gat_layer(h, srcs, dsts, eas, srcs_p, dsts_p, meta, fz,
               Wl, bl, Wr, br, We, att, bc):
    xl = _proj(h, Wl, bl)
    xr = _proj(h, Wr, br)
    gl = _gather_rows(xl, srcs, HC)
    gr = _gather_rows(xr, dsts, HC)
    A = (att[:, :, None] * jnp.eye(H, dtype=jnp.float32)[:, None, :])
    A = A.reshape(HC, H)
    WeP = jnp.pad(We, ((0, 1), (0, 0)))
    alpha = _alpha(gl, gr, eas.reshape(4, E), WeP, A)
    gmax = jnp.max(alpha, axis=0)
    a16 = jnp.pad(alpha, ((0, AGW), (0, 12))).reshape(-1)
    fz2 = jnp.concatenate([fz, jnp.pad(gmax, (0, 12),
                                       constant_values=1e30)])
    wsumf, denf = _aggregate(xl, srcs_p, dsts_p, a16, meta, fz2)
    wsum = wsumf.reshape(NPADN, HC)[:N].reshape(N, H, C)
    den = denf.reshape(NPADN, 16)[:N, :H]
    out = jnp.where(den[:, :, None] > 0.0,
                    wsum / (den[:, :, None] + 1e-16), 0.0)
    return out.reshape(N, HC) + bc


def kernel(x, edge_index, edge_attr, batch, Wl0, bl0, Wr0, br0, We0, att0,
           bc0, Wl1, bl1, Wr1, br1, We1, att1, bc1, ln_g, ln_b, W1, b1, W2,
           b2, W3, b3):
    src, dst = edge_index[0], edge_index[1]
    eye = jnp.eye(16, dtype=jnp.int32)
    iota = jnp.arange(16, dtype=jnp.int32)
    fz = jnp.zeros((16,), jnp.float32)
    perm, offs = _sort_edges(dst, eye, iota)
    eatf = jnp.pad(edge_attr.T, ((0, 1), (0, 0))).reshape(-1)
    srcs, dsts, eas = _permute_payload(perm, src, dst, eatf)
    srcs_p = jnp.pad(srcs, (0, AGW))
    dsts_p = jnp.pad(dsts, (0, AGW))
    bnd = offs[jnp.arange(33, dtype=jnp.int32) * NT]
    meta = jnp.zeros((32, 8), jnp.int32)
    meta = meta.at[:, 0].set(bnd[:-1]).at[:, 1].set(bnd[1:]).reshape(-1)

    h = _gat_layer(x, srcs, dsts, eas, srcs_p, dsts_p, meta, fz,
                   Wl0, bl0, Wr0, br0, We0, att0, bc0)
    h = jax.nn.relu(h)
    mu = jnp.mean(h, axis=-1, keepdims=True)
    var = jnp.var(h, axis=-1, keepdims=True)
    h = (h - mu) / jnp.sqrt(var + 1e-5) * ln_g + ln_b
    h2 = _gat_layer(h, srcs, dsts, eas, srcs_p, dsts_p, meta, fz,
                    Wl1, bl1, Wr1, br1, We1, att1, bc1)
    emb = h2
    h2 = jax.nn.relu(h2)
    pooled = jax.ops.segment_max(h2, batch, num_segments=NG)
    z = jax.nn.relu(pooled @ W1 + b1)
    z = jax.nn.relu(z @ W2 + b2)
    logits = z @ W3 + b3
    logp = jax.nn.log_softmax(logits, axis=1)
    return (emb, logp)


# speedup vs baseline: 1.3364x; 1.1146x over previous
import functools

import jax
import jax.numpy as jnp
from jax import lax
from jax.experimental import pallas as pl
from jax.experimental.pallas import tpu as pltpu
from jax.experimental.pallas import tpu_sc as plsc

N = 10000
E = 320000
H = 4
C = 128
HC = H * C
NG = 16

NWORK = 32
PER_W = E // NWORK
GW = 80
NWIN = PER_W // GW

NPADN = 10016
NT = NPADN // 32
NB = 10256
SW = 128
NSW = E // SW
AGW = 128
EPAD = E + AGW


def _proj_kernel(x_ref, w_ref, b_ref, o_ref):
    o_ref[...] = jnp.dot(x_ref[...], w_ref[...],
                         preferred_element_type=jnp.float32) + b_ref[...]


def _proj(x, w, b):
    m, _ = x.shape
    n = w.shape[1]
    return pl.pallas_call(
        _proj_kernel,
        out_shape=jax.ShapeDtypeStruct((m, n), jnp.float32),
    )(x, w, b[None, :])


def _gather_body(table_hbm, idx_hbm, out_hbm,
                 idx0, idx1, rows0, rows1, g0, g1, w0, w1):
    wid = lax.axis_index("s") * 2 + lax.axis_index("c")
    base = wid * PER_W
    bufs = ((idx0, rows0, g0, w0), (idx1, rows1, g1, w1))

    def start(k, b):
        idxv, rows, gsem, wsem = bufs[b]

        @pl.when(k >= 2)
        def _():
            pltpu.make_async_copy(rows, out_hbm.at[pl.ds(0, GW)],
                                  wsem).wait()
        off = pl.multiple_of(base + k * GW, 8)
        pltpu.sync_copy(idx_hbm.at[pl.ds(off, GW)], idxv.at[0])
        pltpu.async_copy(table_hbm.at[idxv.at[0]], rows, gsem)

    def process(k, b):
        idxv, rows, gsem, wsem = bufs[b]
        valid = k < NWIN

        @pl.when(valid)
        def _():
            pltpu.make_async_copy(table_hbm.at[idxv.at[0]], rows,
                                  gsem).wait()
            off = pl.multiple_of(base + k * GW, 8)
            pltpu.async_copy(rows, out_hbm.at[pl.ds(off, GW)], wsem)

        @pl.when(k + 2 < NWIN)
        def _():
            start(k + 2, b)

    start(0, 0)
    start(1, 1)

    @pl.loop(0, (NWIN + 1) // 2)
    def _(k2):
        process(k2 * 2, 0)
        process(k2 * 2 + 1, 1)

    pltpu.make_async_copy(rows0, out_hbm.at[pl.ds(0, GW)], w0).wait()
    pltpu.make_async_copy(rows1, out_hbm.at[pl.ds(0, GW)], w1).wait()


def _gather_rows(table, idx, d):
    mesh = plsc.VectorSubcoreMesh(core_axis_name="c", subcore_axis_name="s")
    f = pl.kernel(
        _gather_body,
        out_type=jax.ShapeDtypeStruct((E, d), jnp.float32),
        mesh=mesh,
        scratch_types=[
            pltpu.VMEM((1, GW), jnp.int32),
            pltpu.VMEM((1, GW), jnp.int32),
            pltpu.VMEM((GW, d), jnp.float32),
            pltpu.VMEM((GW, d), jnp.float32),
            pltpu.SemaphoreType.DMA,
            pltpu.SemaphoreType.DMA,
            pltpu.SemaphoreType.DMA,
            pltpu.SemaphoreType.DMA,
        ],
        name=f"sc_gather_{d}",
    )
    return f(table, idx)


def _sort_body(dst_hbm, eye_hbm, iota_hbm, perm_hbm, offs_hbm,
               histv, tmpv, wbuf, posv, idsv, eyev, iotav,
               hist_sh, base_sh, sem):
    c = lax.axis_index("c")
    s = lax.axis_index("s")

    @pl.when(c == 0)
    def _():
        pltpu.sync_copy(eye_hbm, eyev)
        pltpu.sync_copy(iota_hbm, iotav)
        eyerows = [eyev[l, :] for l in range(16)]
        inc0 = eyerows[0]
        zerov = inc0 * 0
        iov = iotav[...]
        nwin = (NSW - s + 15) // 16

        @pl.loop(0, NB // 16)
        def _(i):
            histv[pl.ds(i * 16, 16)] = zerov

        def hstep(k, carry):
            off = (s + k * 16) * SW
            pltpu.sync_copy(dst_hbm.at[pl.ds(off, SW)], wbuf)
            for a in range(SW // 16):
                dvec = wbuf[pl.ds(a * 16, 16)]
                for l in range(16):
                    d = dvec[l]
                    histv[pl.ds(d, 16)] = histv[pl.ds(d, 16)] + inc0
            return carry

        lax.fori_loop(0, nwin, hstep, 0)
        pltpu.sync_copy(histv, hist_sh.at[pl.ds(s * NB, NB)])
        plsc.subcore_barrier()

        @pl.when(s == 0)
        def _():
            @pl.loop(0, NB // 16)
            def _(i):
                histv[pl.ds(i * 16, 16)] = zerov

            for tt in range(16):
                pltpu.sync_copy(hist_sh.at[pl.ds(tt * NB, NB)], tmpv)

                @pl.loop(0, NB // 16)
                def _(i):
                    histv[pl.ds(i * 16, 16)] = (histv[pl.ds(i * 16, 16)]
                                                + tmpv[pl.ds(i * 16, 16)])

            def scanstep(i, run):
                v = histv[pl.ds(i * 16, 16)]
                acc = run
                exv = zerov
                for l in range(16):
                    exv = exv + eyerows[l] * acc
                    acc = acc + v[l]
                tmpv[pl.ds(i * 16, 16)] = exv
                return acc

            lax.fori_loop(0, NB // 16, scanstep, 0)
            pltpu.sync_copy(tmpv, offs_hbm)
            for tt in range(16):
                pltpu.sync_copy(tmpv, base_sh.at[pl.ds(tt * NB, NB)])
                pltpu.sync_copy(hist_sh.at[pl.ds(tt * NB, NB)], histv)

                @pl.loop(0, NB // 16)
                def _(i):
                    tmpv[pl.ds(i * 16, 16)] = (tmpv[pl.ds(i * 16, 16)]
                                               + histv[pl.ds(i * 16, 16)])
        plsc.subcore_barrier()

        pltpu.sync_copy(base_sh.at[pl.ds(s * NB, NB)], histv)

        def pstep(k, carry):
            off = (s + k * 16) * SW
            pltpu.sync_copy(dst_hbm.at[pl.ds(off, SW)], wbuf)
            for a in range(SW // 16):
                dvec = wbuf[pl.ds(a * 16, 16)]
                pvec = zerov
                for l in range(16):
                    d = dvec[l]
                    bv = histv[pl.ds(d, 16)]
                    histv[pl.ds(d, 16)] = bv + inc0
                    pvec = pvec + eyerows[l] * bv[0]
                posv[0, pl.ds(a * 16, 16)] = pvec
                idsv[0, pl.ds(a * 16, 16)] = iov + (off + a * 16)
            pltpu.sync_copy(idsv.at[0], perm_hbm.at[posv.at[0]])
            return carry

        lax.fori_loop(0, nwin, pstep, 0)


def _sort_edges(dst, eye, iota):
    mesh = plsc.VectorSubcoreMesh(core_axis_name="c", subcore_axis_name="s")
    f = pl.kernel(
        _sort_body,
        out_type=(jax.ShapeDtypeStruct((E,), jnp.int32),
                  jax.ShapeDtypeStruct((NB,), jnp.int32)),
        mesh=mesh,
        scratch_types=[
            pltpu.VMEM((NB,), jnp.int32),
            pltpu.VMEM((NB,), jnp.int32),
            pltpu.VMEM((SW,), jnp.int32),
            pltpu.VMEM((1, SW), jnp.int32),
            pltpu.VMEM((1, SW), jnp.int32),
            pltpu.VMEM((16, 16), jnp.int32),
            pltpu.VMEM((16,), jnp.int32),
            pltpu.VMEM_SHARED((16 * NB,), jnp.int32),
            pltpu.VMEM_SHARED((16 * NB,), jnp.int32),
            pltpu.SemaphoreType.DMA,
        ],
        name="sc_sort_by_dst",
    )
    return f(dst, eye, iota)


def _permute_body(perm_hbm, src_hbm, dst_hbm, eat_hbm,
                  srcs_hbm, dsts_hbm, eas_hbm,
                  permv, idxcv, srcv, dstv, eav, sem):
    wid = lax.axis_index("s") * 2 + lax.axis_index("c")
    nwin = (NSW - wid + 31) // 32

    def step(k, carry):
        off = pl.multiple_of((wid + k * 32) * SW, SW)
        pltpu.sync_copy(perm_hbm.at[pl.ds(off, SW)], permv.at[0])
        pltpu.async_copy(src_hbm.at[permv.at[0]], srcv, sem).wait()
        pltpu.async_copy(dst_hbm.at[permv.at[0]], dstv, sem).wait()
        pltpu.sync_copy(srcv, srcs_hbm.at[pl.ds(off, SW)])
        pltpu.sync_copy(dstv, dsts_hbm.at[pl.ds(off, SW)])
        for cc in range(4):
            for a in range(SW // 16):
                idxcv[0, pl.ds(a * 16, 16)] = (permv[0, pl.ds(a * 16, 16)]
                                               + cc * E)
            pltpu.async_copy(eat_hbm.at[idxcv.at[0]], eav, sem).wait()
            pltpu.sync_copy(eav, eas_hbm.at[pl.ds(pl.multiple_of(cc * E + off, SW), SW)])
        return carry

    lax.fori_loop(0, nwin, step, 0)


def _permute_payload(perm, src, dst, eatf):
    mesh = plsc.VectorSubcoreMesh(core_axis_name="c", subcore_axis_name="s")
    f = pl.kernel(
        _permute_body,
        out_type=(jax.ShapeDtypeStruct((E,), jnp.int32),
                  jax.ShapeDtypeStruct((E,), jnp.int32),
                  jax.ShapeDtypeStruct((4 * E,), jnp.float32)),
        mesh=mesh,
        scratch_types=[
            pltpu.VMEM((1, SW), jnp.int32),
            pltpu.VMEM((1, SW), jnp.int32),
            pltpu.VMEM((SW,), jnp.int32),
            pltpu.VMEM((SW,), jnp.int32),
            pltpu.VMEM((SW,), jnp.float32),
            pltpu.SemaphoreType.DMA,
        ],
        name="sc_permute_payload",
    )
    return f(perm, src, dst, eatf)


def _alpha_kernel(gl_ref, gr_ref, ea_ref, we_ref, a_ref, o_ref):
    e = lax.dot_general(ea_ref[...], we_ref[...],
                        dimension_numbers=(((0,), (0,)), ((), ())),
                        preferred_element_type=jnp.float32)
    m = gl_ref[...] + gr_ref[...] + e
    m = jnp.where(m >= 0.0, m, 0.2 * m)
    o_ref[...] = jnp.dot(m, a_ref[...], preferred_element_type=jnp.float32)


def _alpha(gl, gr, easT, WeP, A):
    BE = 2048
    grid = (E + BE - 1) // BE
    return pl.pallas_call(
        _alpha_kernel,
        out_shape=jax.ShapeDtypeStruct((E, H), jnp.float32),
        grid=(grid,),
        in_specs=[
            pl.BlockSpec((BE, HC), lambda i: (i, 0)),
            pl.BlockSpec((BE, HC), lambda i: (i, 0)),
            pl.BlockSpec((4, BE), lambda i: (0, i)),
            pl.BlockSpec((4, HC), lambda i: (0, 0)),
            pl.BlockSpec((HC, H), lambda i: (0, 0)),
        ],
        out_specs=pl.BlockSpec((BE, H), lambda i: (i, 0)),
    )(gl, gr, easT, WeP, A)


AGW2 = 96
DWP = AGW2 + 16


def _agg_body(xl_hbm, srcs_hbm, dsts_hbm, p16_hbm, meta_hbm, fz_hbm,
              wsum_hbm, den_hbm,
              metav, fzv, srcw0, srcw1, rows0, rows1, dstwf, pvff,
              accv, denv, gsem0, gsem1, osem):
    wid = lax.axis_index("s") * 2 + lax.axis_index("c")
    pltpu.sync_copy(fz_hbm, fzv)
    zerof = fzv[pl.ds(0, 16)]
    gmaxv = fzv[pl.ds(16, 16)]
    pltpu.sync_copy(meta_hbm.at[pl.ds(pl.multiple_of(wid * 8, 8), 8)],
                    metav.at[pl.ds(0, 8)])
    mv = metav[pl.ds(0, 16)]
    estart = mv[0]
    eend = mv[1]
    nlo = wid * NT
    astart = pl.multiple_of((estart // 8) * 8, 8)
    nwin = (eend - astart + AGW2 - 1) // AGW2

    @pl.loop(0, NT)
    def _(i):
        denv[pl.ds(i * 16, 16)] = zerof

    bufs = ((srcw0, rows0, gsem0, 0), (srcw1, rows1, gsem1, 1))

    def start(w, b):
        srcw, rows, gsem, bi = bufs[b]
        base = pl.multiple_of(astart + w * AGW2, 8)
        pltpu.sync_copy(srcs_hbm.at[pl.ds(base, AGW2)], srcw.at[0])
        pltpu.async_copy(dsts_hbm.at[pl.ds(base, AGW2)],
                         dstwf.at[pl.ds(bi * DWP, AGW2)], gsem)
        pltpu.async_copy(
            p16_hbm.at[pl.ds(pl.multiple_of(base * 16, 128), AGW2 * 16)],
            pvff.at[pl.ds(bi * AGW2 * 16, AGW2 * 16)], gsem)
        pltpu.async_copy(xl_hbm.at[srcw.at[0]], rows, gsem)

    @pl.when(nwin > 0)
    def _():
        start(0, 0)

    @pl.when(nwin > 1)
    def _():
        start(1, 1)

    def process(w, b, carry):
        srcw, rows, gsem, bi = bufs[b]
        valid = w < nwin

        base = pl.multiple_of(astart + w * AGW2, 8)

        @pl.when(valid)
        def _():
            pltpu.make_async_copy(
                dsts_hbm.at[pl.ds(base, AGW2)],
                dstwf.at[pl.ds(bi * DWP, AGW2)], gsem).wait()
            pltpu.make_async_copy(
                p16_hbm.at[pl.ds(pl.multiple_of(base * 16, 128), AGW2 * 16)],
                pvff.at[pl.ds(bi * AGW2 * 16, AGW2 * 16)], gsem).wait()
            pltpu.make_async_copy(xl_hbm.at[srcw.at[0]], rows, gsem).wait()
        jlo = jnp.maximum(0, estart - base)
        jhi = jnp.minimum(AGW2, eend - base)
        jhi = jnp.where(valid, jhi, jlo)
        jhi = jnp.maximum(jlo, jhi)

        def edge_step(j, ecarry):
            cur, flip, cnt = ecarry[0], ecarry[1], ecarry[2]
            accs = ecarry[3:]
            d = dstwf[pl.ds(bi * DWP + j, 16)][0]
            av = pvff[pl.ds(bi * AGW2 * 16 + j * 16, 16)]
            pev = jnp.exp(av - gmaxv)
            is_new = d != cur

            @pl.when(is_new)
            def _():
                @pl.when(cnt > 0)
                def _():
                    pltpu.make_async_copy(
                        accv.at[pl.ds(0, HC)],
                        wsum_hbm.at[pl.ds(0, HC)], osem).wait()
                fo = pl.multiple_of(flip, 8)
                for k in range(HC // 16):
                    accv[pl.ds(fo + k * 16, 16)] = accs[k]
                pltpu.async_copy(
                    accv.at[pl.ds(fo, HC)],
                    wsum_hbm.at[pl.ds(pl.multiple_of(cur * HC, 8), HC)],
                    osem)

            keepf = jnp.where(is_new, 0.0, 1.0)
            cur = jnp.where(is_new, d, cur)
            flip = jnp.where(is_new, HC - flip, flip)
            cnt = cnt + jnp.where(is_new, 1, 0)
            doff = (d - nlo) * 16
            denv[pl.ds(doff, 16)] = denv[pl.ds(doff, 16)] + pev
            ws = (pev[0], pev[1], pev[2], pev[3])
            naccs = tuple(
                accs[k] * keepf + rows[j, pl.ds(k * 16, 16)] * ws[k // 8]
                for k in range(HC // 16))
            return (cur, flip, cnt) + naccs

        carry = lax.fori_loop(jlo, jhi, edge_step, carry)

        @pl.when(w + 2 < nwin)
        def _():
            start(w + 2, b)
        return carry

    carry0 = (nlo, 0, 0) + tuple(zerof for _ in range(HC // 16))

    def pair(w2, carry):
        for b in (0, 1):
            carry = process(w2 * 2 + b, b, carry)
        return carry

    carry = lax.fori_loop(0, (nwin + 1) // 2, pair, carry0)
    cur, flip, cnt = carry[0], carry[1], carry[2]
    accs = carry[3:]

    @pl.when(cnt > 0)
    def _():
        pltpu.make_async_copy(accv.at[pl.ds(0, HC)],
                              wsum_hbm.at[pl.ds(0, HC)], osem).wait()
    fo = pl.multiple_of(flip, 8)
    for k in range(HC // 16):
        accv[pl.ds(fo + k * 16, 16)] = accs[k]
    pltpu.sync_copy(accv.at[pl.ds(fo, HC)],
                    wsum_hbm.at[pl.ds(pl.multiple_of(cur * HC, 8), HC)])
    pltpu.sync_copy(denv, den_hbm.at[pl.ds(
        pl.multiple_of(wid * NT * 16, 16), NT * 16)])


def _aggregate(xl, srcs_p, dsts_p, p16f, meta, fz):
    mesh = plsc.VectorSubcoreMesh(core_axis_name="c", subcore_axis_name="s")
    f = pl.kernel(
        _agg_body,
        out_type=(jax.ShapeDtypeStruct((NPADN * HC,), jnp.float32),
                  jax.ShapeDtypeStruct((NPADN * 16,), jnp.float32)),
        mesh=mesh,
        scratch_types=[
            pltpu.VMEM((16,), jnp.int32),
            pltpu.VMEM((32,), jnp.float32),
            pltpu.VMEM((1, AGW2), jnp.int32),
            pltpu.VMEM((1, AGW2), jnp.int32),
            pltpu.VMEM((AGW2, HC), jnp.float32),
            pltpu.VMEM((AGW2, HC), jnp.float32),
            pltpu.VMEM((2 * DWP,), jnp.int32),
            pltpu.VMEM((2 * AGW2 * 16,), jnp.float32),
            pltpu.VMEM((2 * HC,), jnp.float32),
            pltpu.VMEM((NT * 16,), jnp.float32),
            pltpu.SemaphoreType.DMA,
            pltpu.SemaphoreType.DMA,
            pltpu.SemaphoreType.DMA,
        ],
        name="sc_sorted_agg",
    )
    return f(xl, srcs_p, dsts_p, p16f, meta, fz)


def _gat_layer(h, srcs, dsts, eas, srcs_p, dsts_p, meta, fz,
               Wl, bl, Wr, br, We, att, bc):
    xl = _proj(h, Wl, bl)
    xr = _proj(h, Wr, br)
    gl = _gather_rows(xl, srcs, HC)
    gr = _gather_rows(xr, dsts, HC)
    A = (att[:, :, None] * jnp.eye(H, dtype=jnp.float32)[:, None, :])
    A = A.reshape(HC, H)
    WeP = jnp.pad(We, ((0, 1), (0, 0)))
    alpha = _alpha(gl, gr, eas.reshape(4, E), WeP, A)
    gmax = jnp.max(alpha, axis=0)
    a16 = jnp.pad(alpha, ((0, AGW), (0, 12))).reshape(-1)
    fz2 = jnp.concatenate([fz, jnp.pad(gmax, (0, 12),
                                       constant_values=1e30)])
    wsumf, denf = _aggregate(xl, srcs_p, dsts_p, a16, meta, fz2)
    wsum = wsumf.reshape(NPADN, HC)[:N].reshape(N, H, C)
    den = denf.reshape(NPADN, 16)[:N, :H]
    out = jnp.where(den[:, :, None] > 0.0,
                    wsum / (den[:, :, None] + 1e-16), 0.0)
    return out.reshape(N, HC) + bc


def kernel(x, edge_index, edge_attr, batch, Wl0, bl0, Wr0, br0, We0, att0,
           bc0, Wl1, bl1, Wr1, br1, We1, att1, bc1, ln_g, ln_b, W1, b1, W2,
           b2, W3, b3):
    src, dst = edge_index[0], edge_index[1]
    eye = jnp.eye(16, dtype=jnp.int32)
    iota = jnp.arange(16, dtype=jnp.int32)
    fz = jnp.zeros((16,), jnp.float32)
    perm, offs = _sort_edges(dst, eye, iota)
    eatf = jnp.pad(edge_attr.T, ((0, 1), (0, 0))).reshape(-1)
    srcs, dsts, eas = _permute_payload(perm, src, dst, eatf)
    srcs_p = jnp.pad(srcs, (0, AGW))
    dsts_p = jnp.pad(dsts, (0, AGW))
    bnd = offs[jnp.arange(33, dtype=jnp.int32) * NT]
    meta = jnp.zeros((32, 8), jnp.int32)
    meta = meta.at[:, 0].set(bnd[:-1]).at[:, 1].set(bnd[1:]).reshape(-1)

    h = _gat_layer(x, srcs, dsts, eas, srcs_p, dsts_p, meta, fz,
                   Wl0, bl0, Wr0, br0, We0, att0, bc0)
    h = jax.nn.relu(h)
    mu = jnp.mean(h, axis=-1, keepdims=True)
    var = jnp.var(h, axis=-1, keepdims=True)
    h = (h - mu) / jnp.sqrt(var + 1e-5) * ln_g + ln_b
    h2 = _gat_layer(h, srcs, dsts, eas, srcs_p, dsts_p, meta, fz,
                    Wl1, bl1, Wr1, br1, We1, att1, bc1)
    emb = h2
    h2 = jax.nn.relu(h2)
    pooled = jax.ops.segment_max(h2, batch, num_segments=NG)
    z = jax.nn.relu(pooled @ W1 + b1)
    z = jax.nn.relu(z @ W2 + b2)
    logits = z @ W3 + b3
    logp = jax.nn.log_softmax(logits, axis=1)
    return (emb, logp)


# speedup vs baseline: 1.3608x; 1.0183x over previous
import functools

import jax
import jax.numpy as jnp
from jax import lax
from jax.experimental import pallas as pl
from jax.experimental.pallas import tpu as pltpu
from jax.experimental.pallas import tpu_sc as plsc

N = 10000
E = 320000
H = 4
C = 128
HC = H * C
NG = 16

NWORK = 32
PER_W = E // NWORK
GW = 80
NWIN = PER_W // GW

NPADN = 10016
NT = NPADN // 32
NB = 10256
SW = 128
NSW = E // SW
AGW = 128
EPAD = E + AGW


def _proj_kernel(x_ref, w_ref, b_ref, o_ref):
    o_ref[...] = jnp.dot(x_ref[...], w_ref[...],
                         preferred_element_type=jnp.float32) + b_ref[...]


def _proj(x, w, b):
    m, _ = x.shape
    n = w.shape[1]
    return pl.pallas_call(
        _proj_kernel,
        out_shape=jax.ShapeDtypeStruct((m, n), jnp.float32),
    )(x, w, b[None, :])


def _gather_body(table_hbm, idx_hbm, out_hbm,
                 idx0, idx1, rows0, rows1, g0, g1, w0, w1):
    wid = lax.axis_index("s") * 2 + lax.axis_index("c")
    base = wid * PER_W
    bufs = ((idx0, rows0, g0, w0), (idx1, rows1, g1, w1))

    def start(k, b):
        idxv, rows, gsem, wsem = bufs[b]

        @pl.when(k >= 2)
        def _():
            pltpu.make_async_copy(rows, out_hbm.at[pl.ds(0, GW)],
                                  wsem).wait()
        off = pl.multiple_of(base + k * GW, 8)
        pltpu.sync_copy(idx_hbm.at[pl.ds(off, GW)], idxv.at[0])
        pltpu.async_copy(table_hbm.at[idxv.at[0]], rows, gsem)

    def process(k, b):
        idxv, rows, gsem, wsem = bufs[b]
        valid = k < NWIN

        @pl.when(valid)
        def _():
            pltpu.make_async_copy(table_hbm.at[idxv.at[0]], rows,
                                  gsem).wait()
            off = pl.multiple_of(base + k * GW, 8)
            pltpu.async_copy(rows, out_hbm.at[pl.ds(off, GW)], wsem)

        @pl.when(k + 2 < NWIN)
        def _():
            start(k + 2, b)

    start(0, 0)
    start(1, 1)

    @pl.loop(0, (NWIN + 1) // 2)
    def _(k2):
        process(k2 * 2, 0)
        process(k2 * 2 + 1, 1)

    pltpu.make_async_copy(rows0, out_hbm.at[pl.ds(0, GW)], w0).wait()
    pltpu.make_async_copy(rows1, out_hbm.at[pl.ds(0, GW)], w1).wait()


def _gather_rows(table, idx, d):
    mesh = plsc.VectorSubcoreMesh(core_axis_name="c", subcore_axis_name="s")
    f = pl.kernel(
        _gather_body,
        out_type=jax.ShapeDtypeStruct((E, d), jnp.float32),
        mesh=mesh,
        scratch_types=[
            pltpu.VMEM((1, GW), jnp.int32),
            pltpu.VMEM((1, GW), jnp.int32),
            pltpu.VMEM((GW, d), jnp.float32),
            pltpu.VMEM((GW, d), jnp.float32),
            pltpu.SemaphoreType.DMA,
            pltpu.SemaphoreType.DMA,
            pltpu.SemaphoreType.DMA,
            pltpu.SemaphoreType.DMA,
        ],
        name=f"sc_gather_{d}",
    )
    return f(table, idx)


def _sort_body(dst_hbm, eye_hbm, iota_hbm, perm_hbm, offs_hbm,
               histv, tmpv, wbuf, posv, idsv, eyev, iotav,
               hist_sh, base_sh, sem):
    c = lax.axis_index("c")
    s = lax.axis_index("s")

    @pl.when(c == 0)
    def _():
        pltpu.sync_copy(eye_hbm, eyev)
        pltpu.sync_copy(iota_hbm, iotav)
        eyerows = [eyev[l, :] for l in range(16)]
        inc0 = eyerows[0]
        zerov = inc0 * 0
        iov = iotav[...]
        nwin = (NSW - s + 15) // 16

        @pl.loop(0, NB // 16)
        def _(i):
            histv[pl.ds(i * 16, 16)] = zerov

        def hstep(k, carry):
            off = (s + k * 16) * SW
            pltpu.sync_copy(dst_hbm.at[pl.ds(off, SW)], wbuf)
            for a in range(SW // 16):
                dvec = wbuf[pl.ds(a * 16, 16)]
                for l in range(16):
                    d = dvec[l]
                    histv[pl.ds(d, 16)] = histv[pl.ds(d, 16)] + inc0
            return carry

        lax.fori_loop(0, nwin, hstep, 0)
        pltpu.sync_copy(histv, hist_sh.at[pl.ds(s * NB, NB)])
        plsc.subcore_barrier()

        @pl.when(s == 0)
        def _():
            @pl.loop(0, NB // 16)
            def _(i):
                histv[pl.ds(i * 16, 16)] = zerov

            for tt in range(16):
                pltpu.sync_copy(hist_sh.at[pl.ds(tt * NB, NB)], tmpv)

                @pl.loop(0, NB // 16)
                def _(i):
                    histv[pl.ds(i * 16, 16)] = (histv[pl.ds(i * 16, 16)]
                                                + tmpv[pl.ds(i * 16, 16)])

            def scanstep(i, run):
                v = histv[pl.ds(i * 16, 16)]
                acc = run
                exv = zerov
                for l in range(16):
                    exv = exv + eyerows[l] * acc
                    acc = acc + v[l]
                tmpv[pl.ds(i * 16, 16)] = exv
                return acc

            lax.fori_loop(0, NB // 16, scanstep, 0)
            pltpu.sync_copy(tmpv, offs_hbm)
            for tt in range(16):
                pltpu.sync_copy(tmpv, base_sh.at[pl.ds(tt * NB, NB)])
                pltpu.sync_copy(hist_sh.at[pl.ds(tt * NB, NB)], histv)

                @pl.loop(0, NB // 16)
                def _(i):
                    tmpv[pl.ds(i * 16, 16)] = (tmpv[pl.ds(i * 16, 16)]
                                               + histv[pl.ds(i * 16, 16)])
        plsc.subcore_barrier()

        pltpu.sync_copy(base_sh.at[pl.ds(s * NB, NB)], histv)

        def pstep(k, carry):
            off = (s + k * 16) * SW
            pltpu.sync_copy(dst_hbm.at[pl.ds(off, SW)], wbuf)
            for a in range(SW // 16):
                dvec = wbuf[pl.ds(a * 16, 16)]
                pvec = zerov
                for l in range(16):
                    d = dvec[l]
                    bv = histv[pl.ds(d, 16)]
                    histv[pl.ds(d, 16)] = bv + inc0
                    pvec = pvec + eyerows[l] * bv[0]
                posv[0, pl.ds(a * 16, 16)] = pvec
                idsv[0, pl.ds(a * 16, 16)] = iov + (off + a * 16)
            pltpu.sync_copy(idsv.at[0], perm_hbm.at[posv.at[0]])
            return carry

        lax.fori_loop(0, nwin, pstep, 0)


def _sort_edges(dst, eye, iota):
    mesh = plsc.VectorSubcoreMesh(core_axis_name="c", subcore_axis_name="s")
    f = pl.kernel(
        _sort_body,
        out_type=(jax.ShapeDtypeStruct((E,), jnp.int32),
                  jax.ShapeDtypeStruct((NB,), jnp.int32)),
        mesh=mesh,
        scratch_types=[
            pltpu.VMEM((NB,), jnp.int32),
            pltpu.VMEM((NB,), jnp.int32),
            pltpu.VMEM((SW,), jnp.int32),
            pltpu.VMEM((1, SW), jnp.int32),
            pltpu.VMEM((1, SW), jnp.int32),
            pltpu.VMEM((16, 16), jnp.int32),
            pltpu.VMEM((16,), jnp.int32),
            pltpu.VMEM_SHARED((16 * NB,), jnp.int32),
            pltpu.VMEM_SHARED((16 * NB,), jnp.int32),
            pltpu.SemaphoreType.DMA,
        ],
        name="sc_sort_by_dst",
    )
    return f(dst, eye, iota)


def _permute_body(perm_hbm, src_hbm, dst_hbm, eat_hbm,
                  srcs_hbm, dsts_hbm, eas_hbm,
                  permv, idxcv, srcv, dstv, eav, sem):
    wid = lax.axis_index("s") * 2 + lax.axis_index("c")
    nwin = (NSW - wid + 31) // 32

    def step(k, carry):
        off = pl.multiple_of((wid + k * 32) * SW, SW)
        pltpu.sync_copy(perm_hbm.at[pl.ds(off, SW)], permv.at[0])
        pltpu.async_copy(src_hbm.at[permv.at[0]], srcv, sem).wait()
        pltpu.async_copy(dst_hbm.at[permv.at[0]], dstv, sem).wait()
        pltpu.sync_copy(srcv, srcs_hbm.at[pl.ds(off, SW)])
        pltpu.sync_copy(dstv, dsts_hbm.at[pl.ds(off, SW)])
        for cc in range(4):
            for a in range(SW // 16):
                idxcv[0, pl.ds(a * 16, 16)] = (permv[0, pl.ds(a * 16, 16)]
                                               + cc * E)
            pltpu.async_copy(eat_hbm.at[idxcv.at[0]], eav, sem).wait()
            pltpu.sync_copy(eav, eas_hbm.at[pl.ds(pl.multiple_of(cc * E + off, SW), SW)])
        return carry

    lax.fori_loop(0, nwin, step, 0)


def _permute_payload(perm, src, dst, eatf):
    mesh = plsc.VectorSubcoreMesh(core_axis_name="c", subcore_axis_name="s")
    f = pl.kernel(
        _permute_body,
        out_type=(jax.ShapeDtypeStruct((E,), jnp.int32),
                  jax.ShapeDtypeStruct((E,), jnp.int32),
                  jax.ShapeDtypeStruct((4 * E,), jnp.float32)),
        mesh=mesh,
        scratch_types=[
            pltpu.VMEM((1, SW), jnp.int32),
            pltpu.VMEM((1, SW), jnp.int32),
            pltpu.VMEM((SW,), jnp.int32),
            pltpu.VMEM((SW,), jnp.int32),
            pltpu.VMEM((SW,), jnp.float32),
            pltpu.SemaphoreType.DMA,
        ],
        name="sc_permute_payload",
    )
    return f(perm, src, dst, eatf)


def _alpha_kernel(gl_ref, gr_ref, ea_ref, we_ref, a_ref, o_ref):
    e = lax.dot_general(ea_ref[...], we_ref[...],
                        dimension_numbers=(((0,), (0,)), ((), ())),
                        preferred_element_type=jnp.float32)
    m = gl_ref[...] + gr_ref[...] + e
    m = jnp.where(m >= 0.0, m, 0.2 * m)
    o_ref[...] = jnp.dot(m, a_ref[...], preferred_element_type=jnp.float32)


def _alpha(gl, gr, easT, WeP, A):
    BE = 2048
    grid = (E + BE - 1) // BE
    return pl.pallas_call(
        _alpha_kernel,
        out_shape=jax.ShapeDtypeStruct((E, H), jnp.float32),
        grid=(grid,),
        in_specs=[
            pl.BlockSpec((BE, HC), lambda i: (i, 0)),
            pl.BlockSpec((BE, HC), lambda i: (i, 0)),
            pl.BlockSpec((4, BE), lambda i: (0, i)),
            pl.BlockSpec((4, HC), lambda i: (0, 0)),
            pl.BlockSpec((HC, H), lambda i: (0, 0)),
        ],
        out_specs=pl.BlockSpec((BE, H), lambda i: (i, 0)),
    )(gl, gr, easT, WeP, A)


AGW2 = 96
DWP = AGW2 + 16


def _agg_body(xl_hbm, srcs_hbm, dsts_hbm, p16_hbm, meta_hbm, fz_hbm,
              wsum_hbm, den_hbm,
              metav, fzv, srcw0, srcw1, rows0, rows1, dstwf, pvff,
              accv, denv, gsem0, gsem1, osem):
    wid = lax.axis_index("s") * 2 + lax.axis_index("c")
    pltpu.sync_copy(fz_hbm, fzv)
    zerof = fzv[pl.ds(0, 16)]
    gmaxv = fzv[pl.ds(16, 16)]
    pltpu.sync_copy(meta_hbm.at[pl.ds(pl.multiple_of(wid * 8, 8), 8)],
                    metav.at[pl.ds(0, 8)])
    mv = metav[pl.ds(0, 16)]
    estart = mv[0]
    eend = mv[1]
    nlo = wid * NT
    astart = pl.multiple_of((estart // 8) * 8, 8)
    nwin = (eend - astart + AGW2 - 1) // AGW2

    @pl.loop(0, NT)
    def _(i):
        denv[pl.ds(i * 16, 16)] = zerof

    bufs = ((srcw0, rows0, gsem0, 0), (srcw1, rows1, gsem1, 1))

    def start(w, b):
        srcw, rows, gsem, bi = bufs[b]
        base = pl.multiple_of(astart + w * AGW2, 8)
        pltpu.sync_copy(srcs_hbm.at[pl.ds(base, AGW2)], srcw.at[0])
        pltpu.async_copy(dsts_hbm.at[pl.ds(base, AGW2)],
                         dstwf.at[pl.ds(bi * DWP, AGW2)], gsem)
        pltpu.async_copy(
            p16_hbm.at[pl.ds(pl.multiple_of(base * 16, 128), AGW2 * 16)],
            pvff.at[pl.ds(bi * AGW2 * 16, AGW2 * 16)], gsem)
        pltpu.async_copy(xl_hbm.at[srcw.at[0]], rows, gsem)

    @pl.when(nwin > 0)
    def _():
        start(0, 0)

    @pl.when(nwin > 1)
    def _():
        start(1, 1)

    def process(w, b, carry):
        srcw, rows, gsem, bi = bufs[b]
        valid = w < nwin

        base = pl.multiple_of(astart + w * AGW2, 8)

        @pl.when(valid)
        def _():
            pltpu.make_async_copy(
                dsts_hbm.at[pl.ds(base, AGW2)],
                dstwf.at[pl.ds(bi * DWP, AGW2)], gsem).wait()
            pltpu.make_async_copy(
                p16_hbm.at[pl.ds(pl.multiple_of(base * 16, 128), AGW2 * 16)],
                pvff.at[pl.ds(bi * AGW2 * 16, AGW2 * 16)], gsem).wait()
            pltpu.make_async_copy(xl_hbm.at[srcw.at[0]], rows, gsem).wait()
        jlo = jnp.maximum(0, estart - base)
        jhi = jnp.minimum(AGW2, eend - base)
        jhi = jnp.where(valid, jhi, jlo)
        jhi = jnp.maximum(jlo, jhi)

        def edge_step(j, ecarry):
            cur, flip, cnt = ecarry[0], ecarry[1], ecarry[2]
            accs = ecarry[3:]
            d = dstwf[pl.ds(bi * DWP + j, 16)][0]
            av = pvff[pl.ds(bi * AGW2 * 16 + j * 16, 16)]
            pev = jnp.exp(av - gmaxv)
            is_new = d != cur

            @pl.when(is_new)
            def _():
                @pl.when(cnt > 0)
                def _():
                    pltpu.make_async_copy(
                        accv.at[pl.ds(0, HC)],
                        wsum_hbm.at[pl.ds(0, HC)], osem).wait()
                fo = pl.multiple_of(flip, 8)
                for k in range(HC // 16):
                    accv[pl.ds(fo + k * 16, 16)] = accs[k]
                pltpu.async_copy(
                    accv.at[pl.ds(fo, HC)],
                    wsum_hbm.at[pl.ds(pl.multiple_of(cur * HC, 8), HC)],
                    osem)

            keepf = jnp.where(is_new, 0.0, 1.0)
            cur = jnp.where(is_new, d, cur)
            flip = jnp.where(is_new, HC - flip, flip)
            cnt = cnt + jnp.where(is_new, 1, 0)
            doff = (d - nlo) * 16
            denv[pl.ds(doff, 16)] = denv[pl.ds(doff, 16)] + pev
            ws = (pev[0], pev[1], pev[2], pev[3])
            naccs = tuple(
                accs[k] * keepf + rows[j, pl.ds(k * 16, 16)] * ws[k // 8]
                for k in range(HC // 16))
            return (cur, flip, cnt) + naccs

        carry = lax.fori_loop(jlo, jhi, edge_step, carry)

        @pl.when(w + 2 < nwin)
        def _():
            start(w + 2, b)
        return carry

    carry0 = (nlo, 0, 0) + tuple(zerof for _ in range(HC // 16))

    def pair(w2, carry):
        for b in (0, 1):
            carry = process(w2 * 2 + b, b, carry)
        return carry

    carry = lax.fori_loop(0, (nwin + 1) // 2, pair, carry0)
    cur, flip, cnt = carry[0], carry[1], carry[2]
    accs = carry[3:]

    @pl.when(cnt > 0)
    def _():
        pltpu.make_async_copy(accv.at[pl.ds(0, HC)],
                              wsum_hbm.at[pl.ds(0, HC)], osem).wait()
    fo = pl.multiple_of(flip, 8)
    for k in range(HC // 16):
        accv[pl.ds(fo + k * 16, 16)] = accs[k]
    pltpu.sync_copy(accv.at[pl.ds(fo, HC)],
                    wsum_hbm.at[pl.ds(pl.multiple_of(cur * HC, 8), HC)])
    pltpu.sync_copy(denv, den_hbm.at[pl.ds(
        pl.multiple_of(wid * NT * 16, 16), NT * 16)])


def _aggregate(xl, srcs_p, dsts_p, p16f, meta, fz):
    mesh = plsc.VectorSubcoreMesh(core_axis_name="c", subcore_axis_name="s")
    f = pl.kernel(
        _agg_body,
        out_type=(jax.ShapeDtypeStruct((NPADN * HC,), jnp.float32),
                  jax.ShapeDtypeStruct((NPADN * 16,), jnp.float32)),
        mesh=mesh,
        scratch_types=[
            pltpu.VMEM((16,), jnp.int32),
            pltpu.VMEM((32,), jnp.float32),
            pltpu.VMEM((1, AGW2), jnp.int32),
            pltpu.VMEM((1, AGW2), jnp.int32),
            pltpu.VMEM((AGW2, HC), jnp.float32),
            pltpu.VMEM((AGW2, HC), jnp.float32),
            pltpu.VMEM((2 * DWP,), jnp.int32),
            pltpu.VMEM((2 * AGW2 * 16,), jnp.float32),
            pltpu.VMEM((2 * HC,), jnp.float32),
            pltpu.VMEM((NT * 16,), jnp.float32),
            pltpu.SemaphoreType.DMA,
            pltpu.SemaphoreType.DMA,
            pltpu.SemaphoreType.DMA,
        ],
        name="sc_sorted_agg",
    )
    return f(xl, srcs_p, dsts_p, p16f, meta, fz)


def _gat_layer(h, srcs, dsts, eas, srcs_p, dsts_p, meta, fz,
               Wl, bl, Wr, br, We, att, bc):
    xl = _proj(h, Wl, bl)
    xr = _proj(h, Wr, br)
    gl = _gather_rows(xl, srcs, HC)
    gr = _gather_rows(xr, dsts, HC)
    A = (att[:, :, None] * jnp.eye(H, dtype=jnp.float32)[:, None, :])
    A = A.reshape(HC, H)
    WeP = jnp.pad(We, ((0, 1), (0, 0)))
    alpha = _alpha(gl, gr, eas.reshape(4, E), WeP, A)
    gmax = jnp.max(alpha, axis=0)
    a16 = jnp.pad(alpha, ((0, AGW), (0, 12))).reshape(-1)
    fz2 = jnp.concatenate([fz, jnp.pad(gmax, (0, 12),
                                       constant_values=1e30)])
    wsumf, denf = _aggregate(xl, srcs_p, dsts_p, a16, meta, fz2)
    wsum = wsumf.reshape(NPADN, HC)[:N].reshape(N, H, C)
    den = denf.reshape(NPADN, 16)[:N, :H]
    out = jnp.where(den[:, :, None] > 0.0,
                    wsum / (den[:, :, None] + 1e-16), 0.0)
    return out.reshape(N, HC) + bc


BN = 2000


def _head_kernel(h2_ref, b_ref, w1_ref, b1_ref, w2_ref, b2_ref, w3_ref,
                 b3_ref, o_ref, acc_ref):
    i = pl.program_id(0)

    @pl.when(i == 0)
    def _():
        acc_ref[...] = jnp.full((NG, HC), -jnp.inf, jnp.float32)

    hb = jnp.maximum(h2_ref[...], 0.0)
    bcol = b_ref[...]
    parts = []
    for g in range(NG):
        m = jnp.max(jnp.where(bcol == g, hb, -jnp.inf), axis=0)
        parts.append(m[None, :])
    acc_ref[...] = jnp.maximum(acc_ref[...], jnp.concatenate(parts, axis=0))

    @pl.when(i == (N // BN) - 1)
    def _():
        pooled = acc_ref[...]
        z = jnp.maximum(jnp.dot(pooled, w1_ref[...],
                                preferred_element_type=jnp.float32)
                        + b1_ref[...], 0.0)
        z = jnp.maximum(jnp.dot(z, w2_ref[...],
                                preferred_element_type=jnp.float32)
                        + b2_ref[...], 0.0)
        logits = (jnp.dot(z, w3_ref[...],
                          preferred_element_type=jnp.float32) + b3_ref[...])
        ls = logits - jnp.max(logits, axis=1, keepdims=True)
        o_ref[...] = ls - jnp.log(jnp.sum(jnp.exp(ls), axis=1,
                                          keepdims=True))


def _head(h2, batch, W1, b1, W2, b2, W3, b3):
    return pl.pallas_call(
        _head_kernel,
        out_shape=jax.ShapeDtypeStruct((NG, W3.shape[1]), jnp.float32),
        grid=(N // BN,),
        in_specs=[
            pl.BlockSpec((BN, HC), lambda i: (i, 0)),
            pl.BlockSpec((BN, 1), lambda i: (i, 0)),
            pl.BlockSpec(W1.shape, lambda i: (0, 0)),
            pl.BlockSpec((1, b1.shape[0]), lambda i: (0, 0)),
            pl.BlockSpec(W2.shape, lambda i: (0, 0)),
            pl.BlockSpec((1, b2.shape[0]), lambda i: (0, 0)),
            pl.BlockSpec(W3.shape, lambda i: (0, 0)),
            pl.BlockSpec((1, b3.shape[0]), lambda i: (0, 0)),
        ],
        out_specs=pl.BlockSpec((NG, W3.shape[1]), lambda i: (0, 0)),
        scratch_shapes=[pltpu.VMEM((NG, HC), jnp.float32)],
    )(h2, batch[:, None], W1, b1[None, :], W2, b2[None, :], W3, b3[None, :])


def kernel(x, edge_index, edge_attr, batch, Wl0, bl0, Wr0, br0, We0, att0,
           bc0, Wl1, bl1, Wr1, br1, We1, att1, bc1, ln_g, ln_b, W1, b1, W2,
           b2, W3, b3):
    src, dst = edge_index[0], edge_index[1]
    eye = jnp.eye(16, dtype=jnp.int32)
    iota = jnp.arange(16, dtype=jnp.int32)
    fz = jnp.zeros((16,), jnp.float32)
    perm, offs = _sort_edges(dst, eye, iota)
    eatf = jnp.pad(edge_attr.T, ((0, 1), (0, 0))).reshape(-1)
    srcs, dsts, eas = _permute_payload(perm, src, dst, eatf)
    srcs_p = jnp.pad(srcs, (0, AGW))
    dsts_p = jnp.pad(dsts, (0, AGW))
    bnd = offs[jnp.arange(33, dtype=jnp.int32) * NT]
    meta = jnp.zeros((32, 8), jnp.int32)
    meta = meta.at[:, 0].set(bnd[:-1]).at[:, 1].set(bnd[1:]).reshape(-1)

    h = _gat_layer(x, srcs, dsts, eas, srcs_p, dsts_p, meta, fz,
                   Wl0, bl0, Wr0, br0, We0, att0, bc0)
    h = jax.nn.relu(h)
    mu = jnp.mean(h, axis=-1, keepdims=True)
    var = jnp.var(h, axis=-1, keepdims=True)
    h = (h - mu) / jnp.sqrt(var + 1e-5) * ln_g + ln_b
    h2 = _gat_layer(h, srcs, dsts, eas, srcs_p, dsts_p, meta, fz,
                    Wl1, bl1, Wr1, br1, We1, att1, bc1)
    emb = h2
    logp = _head(h2, batch, W1, b1, W2, b2, W3, b3)
    return (emb, logp)
